# baseline (device time: 27687 ns/iter reference)
import jax
import jax.numpy as jnp
from jax import lax
from jax.experimental import pallas as pl
from jax.experimental.pallas import tpu as pltpu

Y_SIZE = 2
X_SIZE = 2


def _body(o_ref, wo_ref, out_ref, wb_ref, part_ref, ysend_ref, yrecv_ref,
          xsend_ref, xrecv_ref, ysend_sems, yrecv_sems, xsend_sems,
          xrecv_sems):
    my_x = lax.axis_index("x")
    my_y = lax.axis_index("y")
    my_z = lax.axis_index("z")
    other_y = 1 - my_y
    other_x = 1 - my_x
    y_nbr = (my_x, other_y, my_z)
    x_nbr = (other_x, my_y, my_z)

    b_sz, s_half, n_out = out_ref.shape
    k = wo_ref.shape[0]
    n_half = n_out // X_SIZE

    def y_rdma(b):
        return pltpu.make_async_remote_copy(
            src_ref=ysend_ref.at[b],
            dst_ref=yrecv_ref.at[b],
            send_sem=ysend_sems.at[b],
            recv_sem=yrecv_sems.at[b],
            device_id=y_nbr,
            device_id_type=pl.DeviceIdType.MESH,
        )

    def x_rdma(b):
        return pltpu.make_async_remote_copy(
            src_ref=xsend_ref.at[b],
            dst_ref=xrecv_ref.at[b],
            send_sem=xsend_sems.at[b],
            recv_sem=xrecv_sems.at[b],
            device_id=x_nbr,
            device_id_type=pl.DeviceIdType.MESH,
        )

    def o_chunk(y_half, b):
        raw = o_ref[b, pl.ds(y_half * s_half, s_half), :, :]
        return jnp.reshape(raw, (s_half, k)).astype(jnp.bfloat16)

    barrier_sem = pltpu.get_barrier_semaphore()
    for nbr in (y_nbr, x_nbr):
        pl.semaphore_signal(barrier_sem, inc=1, device_id=nbr,
                            device_id_type=pl.DeviceIdType.MESH)

    wb_ref[...] = wo_ref[:, pl.ds(my_x * n_half, n_half)].astype(jnp.bfloat16)

    for b in range(b_sz):
        ysend_ref[b, :, :] = lax.dot(
            o_chunk(other_y, b), wb_ref[...],
            preferred_element_type=jnp.float32,
        ).astype(jnp.bfloat16)
        if b == 0:
            pl.semaphore_wait(barrier_sem, 2)
        y_rdma(b).start()

    for b in range(b_sz):
        part_ref[b, :, :] = lax.dot(
            o_chunk(my_y, b), wb_ref[...],
            preferred_element_type=jnp.float32,
        )

    for b in range(b_sz):
        y_rdma(b).wait_recv()
        total = part_ref[b, :, :] + yrecv_ref[b, :, :].astype(jnp.float32)
        out_ref[b, :, pl.ds(my_x * n_half, n_half)] = total
        xsend_ref[b, :, :] = total.astype(jnp.bfloat16)
        x_rdma(b).start()

    for b in range(b_sz):
        x_rdma(b).wait_recv()
        out_ref[b, :, pl.ds(other_x * n_half, n_half)] = xrecv_ref[
            b, :, :
        ].astype(jnp.float32)

    for b in range(b_sz):
        y_rdma(b).wait_send()
        x_rdma(b).wait_send()


def kernel(O, Wo):
    B, S, H, D = O.shape
    K = H * D
    N = Wo.shape[1]
    s_half = S // Y_SIZE
    n_half = N // X_SIZE
    return pl.pallas_call(
        _body,
        out_shape=jax.ShapeDtypeStruct((B, s_half, N), jnp.float32),
        in_specs=[
            pl.BlockSpec(memory_space=pltpu.VMEM),
            pl.BlockSpec(memory_space=pltpu.VMEM),
        ],
        out_specs=pl.BlockSpec(memory_space=pltpu.VMEM),
        scratch_shapes=[
            pltpu.VMEM((K, n_half), jnp.bfloat16),
            pltpu.VMEM((B, s_half, n_half), jnp.float32),
            pltpu.VMEM((B, s_half, n_half), jnp.bfloat16),
            pltpu.VMEM((B, s_half, n_half), jnp.bfloat16),
            pltpu.VMEM((B, s_half, n_half), jnp.bfloat16),
            pltpu.VMEM((B, s_half, n_half), jnp.bfloat16),
            pltpu.SemaphoreType.DMA((B,)),
            pltpu.SemaphoreType.DMA((B,)),
            pltpu.SemaphoreType.DMA((B,)),
            pltpu.SemaphoreType.DMA((B,)),
        ],
        compiler_params=pltpu.CompilerParams(collective_id=0),
    )(O, Wo)


# device time: 27279 ns/iter; 1.0150x vs baseline; 1.0150x over previous
import jax
import jax.numpy as jnp
from jax import lax
from jax.experimental import pallas as pl
from jax.experimental.pallas import tpu as pltpu

Y_SIZE = 2
X_SIZE = 2
ROWS = 128


def _body(o_ref, wo_ref, out_ref, wb_ref, part_ref, ysend_ref, yrecv_ref,
          xsend_ref, xrecv_ref, ysend_sems, yrecv_sems, xsend_sems,
          xrecv_sems):
    my_x = lax.axis_index("x")
    my_y = lax.axis_index("y")
    my_z = lax.axis_index("z")
    other_y = 1 - my_y
    other_x = 1 - my_x
    y_nbr = (my_x, other_y, my_z)
    x_nbr = (other_x, my_y, my_z)

    b_sz, s_half, n_out = out_ref.shape
    k = wo_ref.shape[0]
    n_half = n_out // X_SIZE
    sub = s_half // ROWS

    def pair_rdma(send_ref, recv_ref, send_sems, recv_sems, nbr, b, j):
        return pltpu.make_async_remote_copy(
            src_ref=send_ref.at[b, pl.ds(j * ROWS, ROWS)],
            dst_ref=recv_ref.at[b, pl.ds(j * ROWS, ROWS)],
            send_sem=send_sems.at[b * sub + j],
            recv_sem=recv_sems.at[b * sub + j],
            device_id=nbr,
            device_id_type=pl.DeviceIdType.MESH,
        )

    def y_rdma(b, j):
        return pair_rdma(ysend_ref, yrecv_ref, ysend_sems, yrecv_sems,
                         y_nbr, b, j)

    def x_rdma(b, j):
        return pair_rdma(xsend_ref, xrecv_ref, xsend_sems, xrecv_sems,
                         x_nbr, b, j)

    def o_chunk(y_half, b, j, rows):
        raw = o_ref[b, pl.ds(y_half * s_half + j * rows, rows), :, :]
        return jnp.reshape(raw, (rows, k)).astype(jnp.bfloat16)

    barrier_sem = pltpu.get_barrier_semaphore()
    for nbr in (y_nbr, x_nbr):
        pl.semaphore_signal(barrier_sem, inc=1, device_id=nbr,
                            device_id_type=pl.DeviceIdType.MESH)

    wb_ref[...] = wo_ref[:, pl.ds(my_x * n_half, n_half)].astype(jnp.bfloat16)

    for b in range(b_sz):
        for j in range(sub):
            ysend_ref[b, pl.ds(j * ROWS, ROWS), :] = lax.dot(
                o_chunk(other_y, b, j, ROWS), wb_ref[...],
                preferred_element_type=jnp.float32,
            ).astype(jnp.bfloat16)
            if b == 0 and j == 0:
                pl.semaphore_wait(barrier_sem, 2)
            y_rdma(b, j).start()

    for b in range(b_sz):
        part_ref[b, :, :] = lax.dot(
            o_chunk(my_y, b, 0, s_half), wb_ref[...],
            preferred_element_type=jnp.float32,
        )

    for b in range(b_sz):
        for j in range(sub):
            y_rdma(b, j).wait_recv()
            rows = pl.ds(j * ROWS, ROWS)
            total = part_ref[b, rows, :] + yrecv_ref[b, rows, :].astype(
                jnp.float32
            )
            out_ref[b, rows, pl.ds(my_x * n_half, n_half)] = total
            xsend_ref[b, rows, :] = total.astype(jnp.bfloat16)
            x_rdma(b, j).start()

    for b in range(b_sz):
        for j in range(sub):
            x_rdma(b, j).wait_recv()
        out_ref[b, :, pl.ds(other_x * n_half, n_half)] = xrecv_ref[
            b, :, :
        ].astype(jnp.float32)

    for b in range(b_sz):
        for j in range(sub):
            y_rdma(b, j).wait_send()
            x_rdma(b, j).wait_send()


def kernel(O, Wo):
    B, S, H, D = O.shape
    K = H * D
    N = Wo.shape[1]
    s_half = S // Y_SIZE
    n_half = N // X_SIZE
    n_chunks = B * (s_half // ROWS)
    return pl.pallas_call(
        _body,
        out_shape=jax.ShapeDtypeStruct((B, s_half, N), jnp.float32),
        in_specs=[
            pl.BlockSpec(memory_space=pltpu.VMEM),
            pl.BlockSpec(memory_space=pltpu.VMEM),
        ],
        out_specs=pl.BlockSpec(memory_space=pltpu.VMEM),
        scratch_shapes=[
            pltpu.VMEM((K, n_half), jnp.bfloat16),
            pltpu.VMEM((B, s_half, n_half), jnp.float32),
            pltpu.VMEM((B, s_half, n_half), jnp.bfloat16),
            pltpu.VMEM((B, s_half, n_half), jnp.bfloat16),
            pltpu.VMEM((B, s_half, n_half), jnp.bfloat16),
            pltpu.VMEM((B, s_half, n_half), jnp.bfloat16),
            pltpu.SemaphoreType.DMA((n_chunks,)),
            pltpu.SemaphoreType.DMA((n_chunks,)),
            pltpu.SemaphoreType.DMA((n_chunks,)),
            pltpu.SemaphoreType.DMA((n_chunks,)),
        ],
        compiler_params=pltpu.CompilerParams(collective_id=0),
    )(O, Wo)


# device time: 26456 ns/iter; 1.0465x vs baseline; 1.0311x over previous
import jax
import jax.numpy as jnp
from jax import lax
from jax.experimental import pallas as pl
from jax.experimental.pallas import tpu as pltpu

Y_SIZE = 2
X_SIZE = 2
ROWS = 128


def _body(o_ref, wo_ref, out_ref, wb_ref, ysend_ref, yrecv_ref,
          xsend_ref, xrecv_ref, ysend_sems, yrecv_sems, xsend_sems,
          xrecv_sems):
    my_x = lax.axis_index("x")
    my_y = lax.axis_index("y")
    my_z = lax.axis_index("z")
    other_y = 1 - my_y
    other_x = 1 - my_x
    y_nbr = (my_x, other_y, my_z)
    x_nbr = (other_x, my_y, my_z)

    b_sz, s_half, n_out = out_ref.shape
    k = wo_ref.shape[0]
    n_half = n_out // X_SIZE
    sub = s_half // ROWS

    def pair_rdma(send_ref, recv_ref, send_sems, recv_sems, nbr, b, j):
        return pltpu.make_async_remote_copy(
            src_ref=send_ref.at[b, pl.ds(j * ROWS, ROWS)],
            dst_ref=recv_ref.at[b, pl.ds(j * ROWS, ROWS)],
            send_sem=send_sems.at[b * sub + j],
            recv_sem=recv_sems.at[b * sub + j],
            device_id=nbr,
            device_id_type=pl.DeviceIdType.MESH,
        )

    def y_rdma(b, j):
        return pair_rdma(ysend_ref, yrecv_ref, ysend_sems, yrecv_sems,
                         y_nbr, b, j)

    def x_rdma(b, j):
        return pair_rdma(xsend_ref, xrecv_ref, xsend_sems, xrecv_sems,
                         x_nbr, b, j)

    def o_chunk(y_half, b, j, rows):
        raw = o_ref[b, pl.ds(y_half * s_half + j * rows, rows), :, :]
        return jnp.reshape(raw, (rows, k)).astype(jnp.bfloat16)

    barrier_sem = pltpu.get_barrier_semaphore()
    for nbr in (y_nbr, x_nbr):
        pl.semaphore_signal(barrier_sem, inc=1, device_id=nbr,
                            device_id_type=pl.DeviceIdType.MESH)

    wb_ref[...] = wo_ref[:, pl.ds(my_x * n_half, n_half)].astype(jnp.bfloat16)

    for b in range(b_sz):
        for j in range(sub):
            ysend_ref[b, pl.ds(j * ROWS, ROWS), :] = lax.dot(
                o_chunk(other_y, b, j, ROWS), wb_ref[...],
                preferred_element_type=jnp.float32,
            ).astype(jnp.bfloat16)
            if b == 0 and j == 0:
                pl.semaphore_wait(barrier_sem, 2)
            y_rdma(b, j).start()

    for b in range(b_sz):
        for j in range(sub):
            part = lax.dot(
                o_chunk(my_y, b, j, ROWS), wb_ref[...],
                preferred_element_type=jnp.float32,
            )
            y_rdma(b, j).wait_recv()
            rows = pl.ds(j * ROWS, ROWS)
            total = part + yrecv_ref[b, rows, :].astype(jnp.float32)
            out_ref[b, rows, pl.ds(my_x * n_half, n_half)] = total
            xsend_ref[b, rows, :] = total.astype(jnp.bfloat16)
            x_rdma(b, j).start()

    for b in range(b_sz):
        for j in range(sub):
            x_rdma(b, j).wait_recv()
        out_ref[b, :, pl.ds(other_x * n_half, n_half)] = xrecv_ref[
            b, :, :
        ].astype(jnp.float32)

    for b in range(b_sz):
        for j in range(sub):
            y_rdma(b, j).wait_send()
            x_rdma(b, j).wait_send()


def kernel(O, Wo):
    B, S, H, D = O.shape
    K = H * D
    N = Wo.shape[1]
    s_half = S // Y_SIZE
    n_half = N // X_SIZE
    n_chunks = B * (s_half // ROWS)
    return pl.pallas_call(
        _body,
        out_shape=jax.ShapeDtypeStruct((B, s_half, N), jnp.float32),
        in_specs=[
            pl.BlockSpec(memory_space=pltpu.VMEM),
            pl.BlockSpec(memory_space=pltpu.VMEM),
        ],
        out_specs=pl.BlockSpec(memory_space=pltpu.VMEM),
        scratch_shapes=[
            pltpu.VMEM((K, n_half), jnp.bfloat16),
            pltpu.VMEM((B, s_half, n_half), jnp.bfloat16),
            pltpu.VMEM((B, s_half, n_half), jnp.bfloat16),
            pltpu.VMEM((B, s_half, n_half), jnp.bfloat16),
            pltpu.VMEM((B, s_half, n_half), jnp.bfloat16),
            pltpu.SemaphoreType.DMA((n_chunks,)),
            pltpu.SemaphoreType.DMA((n_chunks,)),
            pltpu.SemaphoreType.DMA((n_chunks,)),
            pltpu.SemaphoreType.DMA((n_chunks,)),
        ],
        compiler_params=pltpu.CompilerParams(collective_id=0),
    )(O, Wo)
